# Initial kernel scaffold; baseline (speedup 1.0000x reference)
import jax
import jax.numpy as jnp
from jax.experimental import pallas as pl


def kernel(x, edge_index, node, input, W_gnn, b_gnn, W_mlp, b_mlp):
    raise NotImplementedError("write your pallas kernel here")


# speedup vs baseline: 13.7897x; 13.7897x over previous
import jax
import jax.numpy as jnp
from jax import lax
from jax.experimental import pallas as pl
from jax.experimental.pallas import tpu as pltpu
from jax.experimental.pallas import tpu_sc as plsc

N = 10000
E = 320000
D = 128
C = 40
B = 2048

NC = 2
NS = 16
NW = NC * NS
L = 16

EPW = E // NW
NPAD = 10240
K = 128
MAXCH = EPW // K + 2
SLOTS = B + K
DUMMY = B
DROWS = 32
BT = B // NS


def _sc_kernel(x_hbm, src_hbm, dst_hbm, node_hbm,
               acc_out, deg_out,
               src_v, dst_v, node_v, smap_v, deg_l, deg_stage,
               csrc2, cslot2, rows_v, iota_v, slotidx_v, degout_v,
               acc_sh, deg_sh, smap_sh, sem):
  cid = lax.axis_index("c")
  sid = lax.axis_index("s")
  wid = sid * NC + cid
  ebase = wid * EPW

  pltpu.sync_copy(src_hbm.at[pl.ds(ebase, EPW)], src_v)
  pltpu.sync_copy(dst_hbm.at[pl.ds(ebase, EPW)], dst_v)
  pltpu.sync_copy(node_hbm, node_v)

  zeros_f = jnp.zeros((L,), jnp.float32)
  ones_f = jnp.ones((L,), jnp.float32)
  iota = lax.iota(jnp.int32, L)

  def zrows(i, _):
    for j in range(D // L):
      rows_v[i, pl.ds(j * L, L)] = zeros_f
    return 0
  lax.fori_loop(0, K, zrows, 0)

  def zdl(i, _):
    for j in range(K // L):
      deg_l[i, pl.ds(j * L, L)] = zeros_f
    return 0
  lax.fori_loop(0, DROWS, zdl, 0)

  for g in range(DROWS // L):
    iota_v[pl.ds(g * L, L)] = g * L + iota

  zbase = sid * (SLOTS // NS)
  pltpu.sync_copy(rows_v, acc_sh.at[pl.ds(zbase, K)])
  pltpu.sync_copy(rows_v.at[pl.ds(0, SLOTS // NS - K)],
                  acc_sh.at[pl.ds(zbase + K, SLOTS // NS - K)])
  pltpu.sync_copy(rows_v.at[pl.ds(0, DROWS // NS)],
                  deg_sh.at[pl.ds(sid * (DROWS // NS), DROWS // NS)])

  @pl.when(sid == 0)
  def _build():
    neg = jnp.full((L,), -1, jnp.int32)
    def zmap(i, _):
      smap_v[pl.ds(i * L, L)] = neg
      return 0
    lax.fori_loop(0, NPAD // L, zmap, 0)
    def setmap(i, _):
      idx = node_v[pl.ds(i * L, L)]
      plsc.store_scatter(smap_v, [idx], i * L + iota)
      return 0
    lax.fori_loop(0, B // L, setmap, 0)
    pltpu.sync_copy(smap_v, smap_sh)

  plsc.subcore_barrier()
  pltpu.sync_copy(smap_sh, smap_v)

  def compact(g, m):
    d = dst_v[pl.ds(g * L, L)]
    s = src_v[pl.ds(g * L, L)]
    slot = plsc.load_gather(smap_v, [d])
    msk = slot >= 0
    mv = jnp.where(msk, 1, 0).astype(jnp.int32)
    inc = plsc.cumsum(mv)
    pos = m + inc - 1
    plsc.store_scatter(csrc2, [pos // K, pos % K], s, mask=msk)
    plsc.store_scatter(cslot2, [pos // K, pos % K], slot, mask=msk)
    plsc.addupdate_scatter(deg_l, [slot // K, slot % K], ones_f, mask=msk)
    return m + jnp.sum(mv)
  m = lax.fori_loop(0, EPW // L, compact, jnp.int32(0))

  for j in range(K // L):
    pos = m + j * L + iota
    plsc.store_scatter(csrc2, [pos // K, pos % K], jnp.zeros((L,), jnp.int32))
    plsc.store_scatter(cslot2, [pos // K, pos % K],
                       jnp.full((L,), DUMMY, jnp.int32))
  nch = (m + (K - 1)) // K

  def rowchunk(ch, _):
    pltpu.async_copy(x_hbm.at[csrc2.at[ch]], rows_v, sem).wait()
    pltpu.sync_copy(rows_v, acc_sh.at[cslot2.at[ch]], add=True)
    return 0
  lax.fori_loop(0, nch, rowchunk, 0)

  pltpu.sync_copy(deg_l, deg_sh.at[iota_v], add=True)

  plsc.subcore_barrier()

  tbase = sid * BT
  pltpu.sync_copy(deg_sh, deg_stage)
  def sexp(g, _):
    idx = node_v[pl.ds(tbase + g * L, L)]
    slot = plsc.load_gather(smap_v, [idx])
    slotidx_v[pl.ds(g * L, L)] = slot
    degout_v[pl.ds(g * L, L)] = plsc.load_gather(
        deg_stage, [slot // K, slot % K])
    return 0
  lax.fori_loop(0, BT // L, sexp, 0)

  pltpu.async_copy(acc_sh.at[slotidx_v], rows_v, sem).wait()
  pltpu.sync_copy(rows_v, acc_out.at[pl.ds(cid * B + tbase, BT)])
  pltpu.sync_copy(degout_v, deg_out.at[pl.ds(cid * B + tbase, BT)])


def _make_sc():
  mesh = plsc.VectorSubcoreMesh(core_axis_name="c", subcore_axis_name="s")
  return pl.kernel(
      _sc_kernel,
      out_type=[jax.ShapeDtypeStruct((NC * B, D), jnp.float32),
                jax.ShapeDtypeStruct((NC * B,), jnp.float32)],
      mesh=mesh,
      compiler_params=pltpu.CompilerParams(needs_layout_passes=False),
      scratch_types=[
          pltpu.VMEM((EPW,), jnp.int32),
          pltpu.VMEM((EPW,), jnp.int32),
          pltpu.VMEM((B,), jnp.int32),
          pltpu.VMEM((NPAD,), jnp.int32),
          pltpu.VMEM((DROWS, K), jnp.float32),
          pltpu.VMEM((DROWS, K), jnp.float32),
          pltpu.VMEM((MAXCH, K), jnp.int32),
          pltpu.VMEM((MAXCH, K), jnp.int32),
          pltpu.VMEM((K, D), jnp.float32),
          pltpu.VMEM((DROWS,), jnp.int32),
          pltpu.VMEM((BT,), jnp.int32),
          pltpu.VMEM((BT,), jnp.float32),
          pltpu.VMEM_SHARED((SLOTS, D), jnp.float32),
          pltpu.VMEM_SHARED((DROWS, K), jnp.float32),
          pltpu.VMEM_SHARED((NPAD,), jnp.int32),
          pltpu.SemaphoreType.DMA,
      ],
  )


def _tc_head(acc_ref, deg_ref, wg_ref, bg_ref, wm_ref, bm_ref, out_ref):
  a = acc_ref[pl.ds(0, B), :] + acc_ref[pl.ds(B, B), :]
  d = deg_ref[0] + deg_ref[1]
  h = a / jnp.maximum(d, 1.0)
  r = jnp.maximum(jnp.dot(h, wg_ref[...],
                          preferred_element_type=jnp.float32) + bg_ref[...], 0.0)
  out_ref[...] = jnp.dot(r, wm_ref[...],
                         preferred_element_type=jnp.float32) + bm_ref[...]


def kernel(x, edge_index, node, input, W_gnn, b_gnn, W_mlp, b_mlp):
  del input
  src = edge_index[0]
  dst = edge_index[1]
  acc, deg = _make_sc()(x, src, dst, node)
  deg3 = deg.reshape(NC, B, 1)
  out = pl.pallas_call(
      _tc_head,
      out_shape=jax.ShapeDtypeStruct((B, C), jnp.float32),
  )(acc, deg3, W_gnn, b_gnn.reshape(1, D), W_mlp, b_mlp.reshape(1, C))
  return out


# speedup vs baseline: 15.6011x; 1.1314x over previous
import jax
import jax.numpy as jnp
from jax import lax
from jax.experimental import pallas as pl
from jax.experimental.pallas import tpu as pltpu
from jax.experimental.pallas import tpu_sc as plsc

N = 10000
E = 320000
D = 128
C = 40
B = 2048

NC = 2
NS = 16
NW = NC * NS
L = 16

EPW = E // NW
NPAD = 10240
K = 128
MAXCH = EPW // K + 2
SLOTS = B + K
DUMMY = B
DROWS = 32
BT = B // NS


def _sc_kernel(x_hbm, ei_hbm, node_hbm,
               acc_out, deg_out,
               src_v, dst_v, node_v, smap_v, deg_l, deg_stage,
               csrc2, cslot2, rows_a, rows_b, iota_v, slotidx_v, degout_v,
               acc_sh, deg_sh, smap_sh, gsem_a, gsem_b, esem):
  cid = lax.axis_index("c")
  sid = lax.axis_index("s")
  wid = sid * NC + cid
  ebase = wid * EPW

  pltpu.async_copy(ei_hbm.at[pl.ds(ebase, EPW)], src_v, esem)
  pltpu.async_copy(ei_hbm.at[pl.ds(E + ebase, EPW)], dst_v, esem)
  pltpu.sync_copy(node_hbm, node_v)

  zeros_f = jnp.zeros((L,), jnp.float32)
  ones_f = jnp.ones((L,), jnp.float32)
  iota = lax.iota(jnp.int32, L)

  def zrows(i, _):
    for j in range(D // L):
      rows_a[i, pl.ds(j * L, L)] = zeros_f
    return 0
  lax.fori_loop(0, K, zrows, 0)

  def zdl(i, _):
    for j in range(K // L):
      deg_l[i, pl.ds(j * L, L)] = zeros_f
    return 0
  lax.fori_loop(0, DROWS, zdl, 0)

  for g in range(DROWS // L):
    iota_v[pl.ds(g * L, L)] = g * L + iota

  zbase = sid * (SLOTS // NS)
  pltpu.sync_copy(rows_a, acc_sh.at[pl.ds(zbase, K)])
  pltpu.sync_copy(rows_a.at[pl.ds(0, SLOTS // NS - K)],
                  acc_sh.at[pl.ds(zbase + K, SLOTS // NS - K)])
  pltpu.sync_copy(rows_a.at[pl.ds(0, DROWS // NS)],
                  deg_sh.at[pl.ds(sid * (DROWS // NS), DROWS // NS)])

  @pl.when(sid == 0)
  def _build():
    neg = jnp.full((L,), -1, jnp.int32)
    def zmap(i, _):
      smap_v[pl.ds(i * L, L)] = neg
      return 0
    lax.fori_loop(0, NPAD // L, zmap, 0)
    def setmap(i, _):
      idx = node_v[pl.ds(i * L, L)]
      plsc.store_scatter(smap_v, [idx], i * L + iota)
      return 0
    lax.fori_loop(0, B // L, setmap, 0)
    pltpu.sync_copy(smap_v, smap_sh)

  plsc.subcore_barrier()
  pltpu.sync_copy(smap_sh, smap_v)

  pltpu.make_async_copy(ei_hbm.at[pl.ds(ebase, EPW)], src_v, esem).wait()
  pltpu.make_async_copy(ei_hbm.at[pl.ds(E + ebase, EPW)], dst_v, esem).wait()

  def compact(g, m):
    d = dst_v[pl.ds(g * L, L)]
    s = src_v[pl.ds(g * L, L)]
    slot = plsc.load_gather(smap_v, [d])
    msk = slot >= 0
    mv = jnp.where(msk, 1, 0).astype(jnp.int32)
    inc = plsc.cumsum(mv)
    pos = m + inc - 1
    plsc.store_scatter(csrc2, [pos // K, pos % K], s, mask=msk)
    plsc.store_scatter(cslot2, [pos // K, pos % K], slot, mask=msk)
    plsc.addupdate_scatter(deg_l, [slot // K, slot % K], ones_f, mask=msk)
    return m + lax.index_in_dim(inc, L - 1, axis=0, keepdims=False)
  m = lax.fori_loop(0, EPW // L, compact, jnp.int32(0))

  for j in range(K // L):
    pos = m + j * L + iota
    plsc.store_scatter(csrc2, [pos // K, pos % K], jnp.zeros((L,), jnp.int32))
    plsc.store_scatter(cslot2, [pos // K, pos % K],
                       jnp.full((L,), DUMMY, jnp.int32))
  nch = (m + (K - 1)) // K

  @pl.when(nch > 0)
  def _prime():
    pltpu.async_copy(x_hbm.at[csrc2.at[0]], rows_a, gsem_a)

  def rowpair(i, _):
    ch0 = 2 * i
    ch1 = ch0 + 1
    pltpu.make_async_copy(x_hbm.at[csrc2.at[ch0]], rows_a, gsem_a).wait()
    @pl.when(ch1 < nch)
    def _g1():
      pltpu.async_copy(x_hbm.at[csrc2.at[ch1]], rows_b, gsem_b)
    pltpu.sync_copy(rows_a, acc_sh.at[cslot2.at[ch0]], add=True)
    @pl.when(ch1 < nch)
    def _s1():
      pltpu.make_async_copy(x_hbm.at[csrc2.at[ch1]], rows_b, gsem_b).wait()
      @pl.when(ch1 + 1 < nch)
      def _g2():
        pltpu.async_copy(x_hbm.at[csrc2.at[ch1 + 1]], rows_a, gsem_a)
      pltpu.sync_copy(rows_b, acc_sh.at[cslot2.at[ch1]], add=True)
    return 0
  lax.fori_loop(0, (nch + 1) // 2, rowpair, 0)

  pltpu.sync_copy(deg_l, deg_sh.at[iota_v], add=True)

  plsc.subcore_barrier()

  tbase = sid * BT
  pltpu.sync_copy(deg_sh, deg_stage)
  def sexp(g, _):
    idx = node_v[pl.ds(tbase + g * L, L)]
    slot = plsc.load_gather(smap_v, [idx])
    slotidx_v[pl.ds(g * L, L)] = slot
    degout_v[pl.ds(g * L, L)] = plsc.load_gather(
        deg_stage, [slot // K, slot % K])
    return 0
  lax.fori_loop(0, BT // L, sexp, 0)

  pltpu.async_copy(acc_sh.at[slotidx_v], rows_a, gsem_a).wait()
  pltpu.sync_copy(rows_a, acc_out.at[pl.ds(cid * B + tbase, BT)])
  pltpu.sync_copy(degout_v, deg_out.at[pl.ds(cid * B + tbase, BT)])


def _make_sc():
  mesh = plsc.VectorSubcoreMesh(core_axis_name="c", subcore_axis_name="s")
  return pl.kernel(
      _sc_kernel,
      out_type=[jax.ShapeDtypeStruct((NC * B, D), jnp.float32),
                jax.ShapeDtypeStruct((NC * B,), jnp.float32)],
      mesh=mesh,
      compiler_params=pltpu.CompilerParams(needs_layout_passes=False),
      scratch_types=[
          pltpu.VMEM((EPW,), jnp.int32),
          pltpu.VMEM((EPW,), jnp.int32),
          pltpu.VMEM((B,), jnp.int32),
          pltpu.VMEM((NPAD,), jnp.int32),
          pltpu.VMEM((DROWS, K), jnp.float32),
          pltpu.VMEM((DROWS, K), jnp.float32),
          pltpu.VMEM((MAXCH, K), jnp.int32),
          pltpu.VMEM((MAXCH, K), jnp.int32),
          pltpu.VMEM((K, D), jnp.float32),
          pltpu.VMEM((K, D), jnp.float32),
          pltpu.VMEM((DROWS,), jnp.int32),
          pltpu.VMEM((BT,), jnp.int32),
          pltpu.VMEM((BT,), jnp.float32),
          pltpu.VMEM_SHARED((SLOTS, D), jnp.float32),
          pltpu.VMEM_SHARED((DROWS, K), jnp.float32),
          pltpu.VMEM_SHARED((NPAD,), jnp.int32),
          pltpu.SemaphoreType.DMA,
          pltpu.SemaphoreType.DMA,
          pltpu.SemaphoreType.DMA,
      ],
  )


def _tc_head(acc_ref, deg_ref, wg_ref, bg_ref, wm_ref, bm_ref, out_ref):
  a = acc_ref[pl.ds(0, B), :] + acc_ref[pl.ds(B, B), :]
  d = deg_ref[0] + deg_ref[1]
  h = a / jnp.maximum(d, 1.0)
  r = jnp.maximum(jnp.dot(h, wg_ref[...],
                          preferred_element_type=jnp.float32) + bg_ref[...], 0.0)
  out_ref[...] = jnp.dot(r, wm_ref[...],
                         preferred_element_type=jnp.float32) + bm_ref[...]


def kernel(x, edge_index, node, input, W_gnn, b_gnn, W_mlp, b_mlp):
  del input
  acc, deg = _make_sc()(x, edge_index.reshape(-1), node)
  deg3 = deg.reshape(NC, B, 1)
  out = pl.pallas_call(
      _tc_head,
      out_shape=jax.ShapeDtypeStruct((B, C), jnp.float32),
  )(acc, deg3, W_gnn, b_gnn.reshape(1, D), W_mlp, b_mlp.reshape(1, C))
  return out


# speedup vs baseline: 15.6051x; 1.0003x over previous
import jax
import jax.numpy as jnp
from jax import lax
from jax.experimental import pallas as pl
from jax.experimental.pallas import tpu as pltpu
from jax.experimental.pallas import tpu_sc as plsc

N = 10000
E = 320000
D = 128
C = 40
B = 2048

NC = 2
NS = 16
NW = NC * NS
L = 16

EPW = E // NW
NPAD = 10240
K = 128
MAXCH = EPW // K + 2
SLOTS = B + K
DUMMY = B
DROWS = 32
BT = B // NS


def _sc_kernel(x_hbm, ei_hbm, node_hbm,
               acc_out, deg_out,
               src_v, dst_v, node_v, smap_v, deg_l, deg_stage,
               csrc2, cslot2, rows_a, rows_b, iota_v, slotidx_v, degout_v,
               acc_sh, deg_sh, smap_sh, gsem_a, gsem_b, esem):
  cid = lax.axis_index("c")
  sid = lax.axis_index("s")
  wid = sid * NC + cid
  ebase = wid * EPW

  pltpu.async_copy(ei_hbm.at[pl.ds(ebase, EPW)], src_v, esem)
  pltpu.async_copy(ei_hbm.at[pl.ds(E + ebase, EPW)], dst_v, esem)
  pltpu.sync_copy(node_hbm, node_v)

  zeros_f = jnp.zeros((L,), jnp.float32)
  ones_f = jnp.ones((L,), jnp.float32)
  iota = lax.iota(jnp.int32, L)

  def zrows(i, _):
    for j in range(D // L):
      rows_a[i, pl.ds(j * L, L)] = zeros_f
    return 0
  lax.fori_loop(0, K, zrows, 0)

  def zdl(i, _):
    for j in range(K // L):
      deg_l[i, pl.ds(j * L, L)] = zeros_f
    return 0
  lax.fori_loop(0, DROWS, zdl, 0)

  for g in range(DROWS // L):
    iota_v[pl.ds(g * L, L)] = g * L + iota

  zbase = sid * (SLOTS // NS)
  pltpu.sync_copy(rows_a, acc_sh.at[pl.ds(zbase, K)])
  pltpu.sync_copy(rows_a.at[pl.ds(0, SLOTS // NS - K)],
                  acc_sh.at[pl.ds(zbase + K, SLOTS // NS - K)])
  pltpu.sync_copy(rows_a.at[pl.ds(0, DROWS // NS)],
                  deg_sh.at[pl.ds(sid * (DROWS // NS), DROWS // NS)])

  @pl.when(sid == 0)
  def _build():
    neg = jnp.full((L,), -1, jnp.int32)
    def zmap(i, _):
      smap_v[pl.ds(i * L, L)] = neg
      return 0
    lax.fori_loop(0, NPAD // L, zmap, 0)
    def setmap(i, _):
      idx = node_v[pl.ds(i * L, L)]
      plsc.store_scatter(smap_v, [idx], i * L + iota)
      return 0
    lax.fori_loop(0, B // L, setmap, 0)
    pltpu.sync_copy(smap_v, smap_sh)

  plsc.subcore_barrier()
  pltpu.sync_copy(smap_sh, smap_v)

  pltpu.make_async_copy(ei_hbm.at[pl.ds(ebase, EPW)], src_v, esem).wait()
  pltpu.make_async_copy(ei_hbm.at[pl.ds(E + ebase, EPW)], dst_v, esem).wait()

  sc_compact = jax.named_scope("sc_compact")
  sc_compact.__enter__()
  def compact(g, m):
    d = dst_v[pl.ds(g * L, L)]
    s = src_v[pl.ds(g * L, L)]
    slot = plsc.load_gather(smap_v, [d])
    msk = slot >= 0
    mv = jnp.where(msk, 1, 0).astype(jnp.int32)
    inc = plsc.cumsum(mv)
    pos = m + inc - 1
    plsc.store_scatter(csrc2, [pos // K, pos % K], s, mask=msk)
    plsc.store_scatter(cslot2, [pos // K, pos % K], slot, mask=msk)
    plsc.addupdate_scatter(deg_l, [slot // K, slot % K], ones_f, mask=msk)
    return m + lax.index_in_dim(inc, L - 1, axis=0, keepdims=False)
  m = lax.fori_loop(0, EPW // L, compact, jnp.int32(0))

  for j in range(K // L):
    pos = m + j * L + iota
    plsc.store_scatter(csrc2, [pos // K, pos % K], jnp.zeros((L,), jnp.int32))
    plsc.store_scatter(cslot2, [pos // K, pos % K],
                       jnp.full((L,), DUMMY, jnp.int32))
  nch = (m + (K - 1)) // K
  sc_compact.__exit__(None, None, None)
  sc_rows = jax.named_scope("sc_rows")
  sc_rows.__enter__()

  @pl.when(nch > 0)
  def _prime():
    pltpu.async_copy(x_hbm.at[csrc2.at[0]], rows_a, gsem_a)

  def rowpair(i, _):
    ch0 = 2 * i
    ch1 = ch0 + 1
    pltpu.make_async_copy(x_hbm.at[csrc2.at[ch0]], rows_a, gsem_a).wait()
    @pl.when(ch1 < nch)
    def _g1():
      pltpu.async_copy(x_hbm.at[csrc2.at[ch1]], rows_b, gsem_b)
    pltpu.sync_copy(rows_a, acc_sh.at[cslot2.at[ch0]], add=True)
    @pl.when(ch1 < nch)
    def _s1():
      pltpu.make_async_copy(x_hbm.at[csrc2.at[ch1]], rows_b, gsem_b).wait()
      @pl.when(ch1 + 1 < nch)
      def _g2():
        pltpu.async_copy(x_hbm.at[csrc2.at[ch1 + 1]], rows_a, gsem_a)
      pltpu.sync_copy(rows_b, acc_sh.at[cslot2.at[ch1]], add=True)
    return 0
  lax.fori_loop(0, (nch + 1) // 2, rowpair, 0)

  pltpu.sync_copy(deg_l, deg_sh.at[iota_v], add=True)
  sc_rows.__exit__(None, None, None)

  plsc.subcore_barrier()

  tbase = sid * BT
  pltpu.sync_copy(deg_sh, deg_stage)
  def sexp(g, _):
    idx = node_v[pl.ds(tbase + g * L, L)]
    slot = plsc.load_gather(smap_v, [idx])
    slotidx_v[pl.ds(g * L, L)] = slot
    degout_v[pl.ds(g * L, L)] = plsc.load_gather(
        deg_stage, [slot // K, slot % K])
    return 0
  lax.fori_loop(0, BT // L, sexp, 0)

  pltpu.async_copy(acc_sh.at[slotidx_v], rows_a, gsem_a).wait()
  pltpu.sync_copy(rows_a, acc_out.at[pl.ds(cid * B + tbase, BT)])
  pltpu.sync_copy(degout_v, deg_out.at[pl.ds(cid * B + tbase, BT)])


def _make_sc():
  mesh = plsc.VectorSubcoreMesh(core_axis_name="c", subcore_axis_name="s")
  return pl.kernel(
      _sc_kernel,
      out_type=[jax.ShapeDtypeStruct((NC * B, D), jnp.float32),
                jax.ShapeDtypeStruct((NC * B,), jnp.float32)],
      mesh=mesh,
      compiler_params=pltpu.CompilerParams(needs_layout_passes=False),
      scratch_types=[
          pltpu.VMEM((EPW,), jnp.int32),
          pltpu.VMEM((EPW,), jnp.int32),
          pltpu.VMEM((B,), jnp.int32),
          pltpu.VMEM((NPAD,), jnp.int32),
          pltpu.VMEM((DROWS, K), jnp.float32),
          pltpu.VMEM((DROWS, K), jnp.float32),
          pltpu.VMEM((MAXCH, K), jnp.int32),
          pltpu.VMEM((MAXCH, K), jnp.int32),
          pltpu.VMEM((K, D), jnp.float32),
          pltpu.VMEM((K, D), jnp.float32),
          pltpu.VMEM((DROWS,), jnp.int32),
          pltpu.VMEM((BT,), jnp.int32),
          pltpu.VMEM((BT,), jnp.float32),
          pltpu.VMEM_SHARED((SLOTS, D), jnp.float32),
          pltpu.VMEM_SHARED((DROWS, K), jnp.float32),
          pltpu.VMEM_SHARED((NPAD,), jnp.int32),
          pltpu.SemaphoreType.DMA,
          pltpu.SemaphoreType.DMA,
          pltpu.SemaphoreType.DMA,
      ],
  )


def _tc_head(acc_ref, deg_ref, wg_ref, bg_ref, wm_ref, bm_ref, out_ref):
  a = acc_ref[pl.ds(0, B), :] + acc_ref[pl.ds(B, B), :]
  d = deg_ref[0] + deg_ref[1]
  h = a / jnp.maximum(d, 1.0)
  r = jnp.maximum(jnp.dot(h, wg_ref[...],
                          preferred_element_type=jnp.float32) + bg_ref[...], 0.0)
  out_ref[...] = jnp.dot(r, wm_ref[...],
                         preferred_element_type=jnp.float32) + bm_ref[...]


def kernel(x, edge_index, node, input, W_gnn, b_gnn, W_mlp, b_mlp):
  del input
  acc, deg = _make_sc()(x, edge_index.reshape(-1), node)
  deg3 = deg.reshape(NC, B, 1)
  out = pl.pallas_call(
      _tc_head,
      out_shape=jax.ShapeDtypeStruct((B, C), jnp.float32),
  )(acc, deg3, W_gnn, b_gnn.reshape(1, D), W_mlp, b_mlp.reshape(1, C))
  return out
